# baseline (speedup 1.0000x reference)
import jax
import jax.numpy as jnp
from jax import lax
from jax.experimental import pallas as pl
from jax.experimental.pallas import tpu as pltpu
from jax.experimental.pallas import tpu_sc as plsc

BATCH = 16384
POOL = 20
EMBED_DIM = 64
NB = 4096
NC, NS, L = 2, 16, 16
NW = NC * NS
COLS_W = BATCH // NW
ITERS = POOL * COLS_W // L


_mesh = plsc.VectorSubcoreMesh(
    core_axis_name="c", subcore_axis_name="s", num_cores=NC, num_subcores=NS
)

LCOLS = NB // NS

_SC_SCRATCH = [
    pltpu.VMEM((NB,), jnp.float32),
    pltpu.VMEM((NB,), jnp.float32),
    pltpu.VMEM((POOL, COLS_W), jnp.int32),
    pltpu.VMEM((POOL, COLS_W), jnp.int32),
    pltpu.VMEM((EMBED_DIM, LCOLS), jnp.float32),
    pltpu.VMEM((EMBED_DIM, LCOLS), jnp.float32),
    pltpu.VMEM((LCOLS,), jnp.float32),
    pltpu.VMEM((L,), jnp.float32),
    pltpu.VMEM_SHARED((NB,), jnp.float32),
    pltpu.VMEM_SHARED((NB,), jnp.float32),
    pltpu.SemaphoreType.DMA,
    pltpu.SemaphoreType.DMA,
    pltpu.SemaphoreType.DMA,
    pltpu.SemaphoreType.DMA,
]

_NACC = 4


def _colsum(tab_v, part_v):
    for g in range(LCOLS // L):
        accs = [jnp.zeros((L,), jnp.float32) for _ in range(_NACC)]
        for r in range(EMBED_DIM):
            accs[r % _NACC] = accs[r % _NACC] + tab_v[r, pl.ds(g * L, L)]
        part_v[pl.ds(g * L, L)] = (accs[0] + accs[1]) + (accs[2] + accs[3])


def _sc_all_body(t0_hbm, t1_hbm, it0_hbm, it1_hbm, out_hbm,
                 lut0, lut1, idx0_v, idx1_v, tab0_v, tab1_v, part_v, acc_v,
                 sh0, sh1, sem0, sem1, sem2, sem3):
    s = lax.axis_index("s")
    wid = s * NC + lax.axis_index("c")
    base = wid * COLS_W
    col0 = s * LCOLS
    cp_t0 = pltpu.async_copy(t0_hbm.at[:, pl.ds(col0, LCOLS)], tab0_v, sem0)
    cp_t1 = pltpu.async_copy(t1_hbm.at[:, pl.ds(col0, LCOLS)], tab1_v, sem1)
    cp_i0 = pltpu.async_copy(it0_hbm.at[:, pl.ds(base, COLS_W)], idx0_v, sem2)
    cp_i1 = pltpu.async_copy(it1_hbm.at[:, pl.ds(base, COLS_W)], idx1_v, sem3)

    cp_t0.wait()
    _colsum(tab0_v, part_v)
    pltpu.sync_copy(part_v, sh0.at[pl.ds(col0, LCOLS)])
    cp_t1.wait()
    _colsum(tab1_v, part_v)
    pltpu.sync_copy(part_v, sh1.at[pl.ds(col0, LCOLS)])

    plsc.subcore_barrier()
    pltpu.sync_copy(sh0, lut0)
    pltpu.sync_copy(sh1, lut1)

    npc = COLS_W // L
    zeros = tuple(jnp.zeros((L,), jnp.float32) for _ in range(_NACC))

    def table_loop(lut, idx_v, accs):
        def row_body(r, accs):
            accs = list(accs)
            for c in range(npc):
                iv = idx_v[r, pl.ds(c * L, L)]
                accs[c % _NACC] = accs[c % _NACC] + plsc.load_gather(lut, [iv])
            return tuple(accs)
        return lax.fori_loop(0, POOL, row_body, accs)

    cp_i0.wait()
    accs = table_loop(lut0, idx0_v, zeros)
    cp_i1.wait()
    accs = table_loop(lut1, idx1_v, accs)
    acc_v[...] = (accs[0] + accs[1]) + (accs[2] + accs[3])
    pltpu.sync_copy(acc_v, out_hbm.at[wid])


_sc_all = pl.kernel(
    _sc_all_body,
    out_type=jax.ShapeDtypeStruct((NW, L), jnp.float32),
    mesh=_mesh,
    scratch_types=_SC_SCRATCH,
    compiler_params=pltpu.CompilerParams(needs_layout_passes=False),
)


def kernel(indices_0, indices_1, table_0, table_1):
    partials = _sc_all(table_0.T, table_1.T, indices_0.T, indices_1.T)
    return jnp.sum(partials) / jnp.float32(BATCH * 2 * EMBED_DIM)


# speedup vs baseline: 1.4829x; 1.4829x over previous
import jax
import jax.numpy as jnp
from jax import lax
from jax.experimental import pallas as pl
from jax.experimental.pallas import tpu as pltpu
from jax.experimental.pallas import tpu_sc as plsc

BATCH = 16384
POOL = 20
EMBED_DIM = 64
NB = 4096
NC, NS, L = 2, 16, 16
NW = NC * NS
COLS_W = BATCH // NW
ITERS = POOL * COLS_W // L


def _rowsum_body(t0_ref, t1_ref, rs_ref):
    rs_ref[0, :] = jnp.sum(t0_ref[...], axis=0)
    rs_ref[1, :] = jnp.sum(t1_ref[...], axis=0)


_rowsum = pl.pallas_call(
    _rowsum_body,
    grid=(1,),
    out_shape=jax.ShapeDtypeStruct((2, NB), jnp.float32),
    in_specs=[
        pl.BlockSpec((EMBED_DIM, NB), lambda i: (0, 0)),
        pl.BlockSpec((EMBED_DIM, NB), lambda i: (0, 0)),
    ],
    out_specs=pl.BlockSpec((2, NB), lambda i: (0, 0)),
)

_mesh = plsc.VectorSubcoreMesh(
    core_axis_name="c", subcore_axis_name="s", num_cores=NC, num_subcores=NS
)

_SC_SCRATCH = [
    pltpu.VMEM((NB,), jnp.float32),
    pltpu.VMEM((NB,), jnp.float32),
    pltpu.VMEM((POOL, COLS_W), jnp.int32),
    pltpu.VMEM((POOL, COLS_W), jnp.int32),
    pltpu.VMEM((L,), jnp.float32),
    pltpu.SemaphoreType.DMA,
    pltpu.SemaphoreType.DMA,
    pltpu.SemaphoreType.DMA,
    pltpu.SemaphoreType.DMA,
]

_NACC = 4


def _sc_gather_sum_body(rs_hbm, it0_hbm, it1_hbm, out_hbm,
                        lut0, lut1, idx0_v, idx1_v, acc_v,
                        sem0, sem1, sem2, sem3):
    wid = lax.axis_index("s") * NC + lax.axis_index("c")
    base = wid * COLS_W
    cp_l0 = pltpu.async_copy(rs_hbm.at[0], lut0, sem0)
    cp_i0 = pltpu.async_copy(it0_hbm.at[:, pl.ds(base, COLS_W)], idx0_v, sem1)
    cp_l1 = pltpu.async_copy(rs_hbm.at[1], lut1, sem2)
    cp_i1 = pltpu.async_copy(it1_hbm.at[:, pl.ds(base, COLS_W)], idx1_v, sem3)

    npc = COLS_W // L
    zeros = tuple(jnp.zeros((L,), jnp.float32) for _ in range(_NACC))

    def table_loop(lut, idx_v, accs):
        def row_body(r, accs):
            accs = list(accs)
            for c in range(npc):
                iv = idx_v[r, pl.ds(c * L, L)]
                accs[c % _NACC] = accs[c % _NACC] + plsc.load_gather(lut, [iv])
            return tuple(accs)
        return plsc.parallel_loop(0, POOL, carry=accs, unroll=2)(row_body)

    cp_l0.wait()
    cp_i0.wait()
    accs = table_loop(lut0, idx0_v, zeros)
    cp_l1.wait()
    cp_i1.wait()
    accs = table_loop(lut1, idx1_v, accs)
    acc_v[...] = (accs[0] + accs[1]) + (accs[2] + accs[3])
    pltpu.sync_copy(acc_v, out_hbm.at[wid])


_sc_gather_sum = pl.kernel(
    _sc_gather_sum_body,
    out_type=jax.ShapeDtypeStruct((NW, L), jnp.float32),
    mesh=_mesh,
    scratch_types=_SC_SCRATCH,
    compiler_params=pltpu.CompilerParams(needs_layout_passes=False),
)


def kernel(indices_0, indices_1, table_0, table_1):
    rs = _rowsum(table_0.T, table_1.T)
    partials = _sc_gather_sum(rs, indices_0.T, indices_1.T)
    return jnp.sum(partials) / jnp.float32(BATCH * 2 * EMBED_DIM)


# speedup vs baseline: 1.5276x; 1.0301x over previous
import jax
import jax.numpy as jnp
from jax import lax
from jax.experimental import pallas as pl
from jax.experimental.pallas import tpu as pltpu
from jax.experimental.pallas import tpu_sc as plsc

BATCH = 16384
POOL = 20
EMBED_DIM = 64
NB = 4096
NC, NS, L = 2, 16, 16
NW = NC * NS
COLS_W = BATCH // NW


def _rowsum_body(t0_ref, t1_ref, rs_ref):
    rs_ref[0, :] = jnp.sum(t0_ref[...], axis=0)
    rs_ref[1, :] = jnp.sum(t1_ref[...], axis=0)


_rowsum = pl.pallas_call(
    _rowsum_body,
    grid=(1,),
    out_shape=jax.ShapeDtypeStruct((2, NB), jnp.float32),
    in_specs=[
        pl.BlockSpec((EMBED_DIM, NB), lambda i: (0, 0)),
        pl.BlockSpec((EMBED_DIM, NB), lambda i: (0, 0)),
    ],
    out_specs=pl.BlockSpec((2, NB), lambda i: (0, 0)),
)

_mesh = plsc.VectorSubcoreMesh(
    core_axis_name="c", subcore_axis_name="s", num_cores=NC, num_subcores=NS
)

_SC_SCRATCH = [
    pltpu.VMEM((NB,), jnp.float32),
    pltpu.VMEM((NB,), jnp.float32),
    pltpu.VMEM((POOL, COLS_W), jnp.int32),
    pltpu.VMEM((POOL, COLS_W), jnp.int32),
    pltpu.VMEM((L,), jnp.float32),
    pltpu.SemaphoreType.DMA,
    pltpu.SemaphoreType.DMA,
    pltpu.SemaphoreType.DMA,
    pltpu.SemaphoreType.DMA,
]

_NACC = 4


def _sc_gather_sum_body(rs_hbm, it0_hbm, it1_hbm, out_hbm,
                        lut0, lut1, idx0_v, idx1_v, acc_v,
                        sem0, sem1, sem2, sem3):
    wid = lax.axis_index("s") * NC + lax.axis_index("c")
    base = wid * COLS_W
    cp_l0 = pltpu.async_copy(rs_hbm.at[0], lut0, sem0)
    cp_i0 = pltpu.async_copy(it0_hbm.at[:, pl.ds(base, COLS_W)], idx0_v, sem1)
    cp_l1 = pltpu.async_copy(rs_hbm.at[1], lut1, sem2)
    cp_i1 = pltpu.async_copy(it1_hbm.at[:, pl.ds(base, COLS_W)], idx1_v, sem3)

    npc = COLS_W // L
    zeros = tuple(jnp.zeros((L,), jnp.float32) for _ in range(_NACC))

    def table_loop(lut, idx_v, accs):
        def row_body(r, accs):
            accs = list(accs)
            for c in range(npc):
                iv = idx_v[r, pl.ds(c * L, L)]
                accs[c % _NACC] = accs[c % _NACC] + plsc.load_gather(lut, [iv])
            return tuple(accs)
        return lax.fori_loop(0, POOL, row_body, accs)

    cp_l0.wait()
    cp_i0.wait()
    accs = table_loop(lut0, idx0_v, zeros)
    cp_l1.wait()
    cp_i1.wait()
    accs = table_loop(lut1, idx1_v, accs)
    acc_v[...] = (accs[0] + accs[1]) + (accs[2] + accs[3])
    pltpu.sync_copy(acc_v, out_hbm.at[wid])


_sc_gather_sum = pl.kernel(
    _sc_gather_sum_body,
    out_type=jax.ShapeDtypeStruct((NW, L), jnp.float32),
    mesh=_mesh,
    scratch_types=_SC_SCRATCH,
    compiler_params=pltpu.CompilerParams(needs_layout_passes=False),
)


def kernel(indices_0, indices_1, table_0, table_1):
    rs = _rowsum(table_0.T, table_1.T)
    partials = _sc_gather_sum(rs, indices_0.T, indices_1.T)
    return jnp.sum(partials) / jnp.float32(BATCH * 2 * EMBED_DIM)
